# baseline (device time: 34767 ns/iter reference)
import jax
import jax.numpy as jnp
from jax import lax
from jax.experimental import pallas as pl
from jax.experimental.pallas import tpu as pltpu

M = 512
D = 512
F = 2048
HALF = D // 4
CHUNK = D // 2


def kernel(x, dy):
    def body(x_ref, dy_ref, out_ref, p_ref, rx_ref, ry_ref,
             send_sems, recv_sems):
        my_k = lax.axis_index("x")
        my_j = lax.axis_index("y")

        p_ref[:, :] = lax.dot_general(
            x_ref[:, :], dy_ref[:, :],
            dimension_numbers=(((0,), (0,)), ((), ())),
            preferred_element_type=jnp.float32,
        )

        barrier_sem = pltpu.get_barrier_semaphore()
        pl.semaphore_signal(
            barrier_sem, inc=1,
            device_id=(1 - my_k, my_j),
            device_id_type=pl.DeviceIdType.MESH,
        )
        pl.semaphore_signal(
            barrier_sem, inc=1,
            device_id=(my_k, 1 - my_j),
            device_id_type=pl.DeviceIdType.MESH,
        )
        pl.semaphore_wait(barrier_sem, 2)

        send_start = (1 - my_k) * CHUNK + my_j * HALF
        rdma_x = pltpu.make_async_remote_copy(
            src_ref=p_ref.at[pl.ds(send_start, HALF), :],
            dst_ref=rx_ref,
            send_sem=send_sems.at[0],
            recv_sem=recv_sems.at[0],
            device_id=(1 - my_k, my_j),
            device_id_type=pl.DeviceIdType.MESH,
        )
        rdma_x.start()
        rdma_x.wait()

        rdma_y = pltpu.make_async_remote_copy(
            src_ref=rx_ref,
            dst_ref=ry_ref,
            send_sem=send_sems.at[1],
            recv_sem=recv_sems.at[1],
            device_id=(my_k, 1 - my_j),
            device_id_type=pl.DeviceIdType.MESH,
        )
        rdma_y.start()
        rdma_y.wait()

        own = my_k * CHUNK
        out_ref[pl.ds(my_j * HALF, HALF), :] = (
            p_ref[pl.ds(own + my_j * HALF, HALF), :] + rx_ref[:, :]
        )
        out_ref[pl.ds((1 - my_j) * HALF, HALF), :] = (
            p_ref[pl.ds(own + (1 - my_j) * HALF, HALF), :] + ry_ref[:, :]
        )

    return pl.pallas_call(
        body,
        out_shape=jax.ShapeDtypeStruct((CHUNK, F), jnp.float32),
        in_specs=[
            pl.BlockSpec(memory_space=pltpu.VMEM),
            pl.BlockSpec(memory_space=pltpu.VMEM),
        ],
        out_specs=pl.BlockSpec(memory_space=pltpu.VMEM),
        scratch_shapes=[
            pltpu.VMEM((D, F), jnp.float32),
            pltpu.VMEM((HALF, F), jnp.float32),
            pltpu.VMEM((HALF, F), jnp.float32),
            pltpu.SemaphoreType.DMA((2,)),
            pltpu.SemaphoreType.DMA((2,)),
        ],
        compiler_params=pltpu.CompilerParams(collective_id=0),
    )(x, dy)


# device time: 25706 ns/iter; 1.3525x vs baseline; 1.3525x over previous
import jax
import jax.numpy as jnp
from jax import lax
from jax.experimental import pallas as pl
from jax.experimental.pallas import tpu as pltpu

M = 512
D = 512
F = 2048
HALF = D // 4
CHUNK = D // 2
NCHUNK = 4
R = HALF // NCHUNK


def _dotT(a, b):
    return lax.dot_general(
        a, b,
        dimension_numbers=(((0,), (0,)), ((), ())),
        preferred_element_type=jnp.float32,
    )


def kernel(x, dy):
    def body(x_ref, dy_ref, out_ref, psend_ref, pown_ref, rx_ref, ry_ref,
             sendx_sems, recvx_sems, sendy_sems, recvy_sems):
        my_k = lax.axis_index("x")
        my_j = lax.axis_index("y")

        barrier_sem = pltpu.get_barrier_semaphore()
        pl.semaphore_signal(
            barrier_sem, inc=1,
            device_id=(1 - my_k, my_j),
            device_id_type=pl.DeviceIdType.MESH,
        )
        pl.semaphore_signal(
            barrier_sem, inc=1,
            device_id=(my_k, 1 - my_j),
            device_id_type=pl.DeviceIdType.MESH,
        )
        pl.semaphore_wait(barrier_sem, 2)

        send_start = (1 - my_k) * CHUNK + my_j * HALF
        psend_ref[:, :] = _dotT(
            x_ref[:, pl.ds(send_start, HALF)], dy_ref[:, :]
        )

        rdmas_x = []
        for c in range(NCHUNK):
            r = pltpu.make_async_remote_copy(
                src_ref=psend_ref.at[pl.ds(c * R, R), :],
                dst_ref=rx_ref.at[pl.ds(c * R, R), :],
                send_sem=sendx_sems.at[c],
                recv_sem=recvx_sems.at[c],
                device_id=(1 - my_k, my_j),
                device_id_type=pl.DeviceIdType.MESH,
            )
            r.start()
            rdmas_x.append(r)

        own = my_k * CHUNK
        pown_ref[:, :] = _dotT(x_ref[:, pl.ds(own, CHUNK)], dy_ref[:, :])

        rdmas_y = []
        for c in range(NCHUNK):
            rdmas_x[c].wait_recv()
            r = pltpu.make_async_remote_copy(
                src_ref=rx_ref.at[pl.ds(c * R, R), :],
                dst_ref=ry_ref.at[pl.ds(c * R, R), :],
                send_sem=sendy_sems.at[c],
                recv_sem=recvy_sems.at[c],
                device_id=(my_k, 1 - my_j),
                device_id_type=pl.DeviceIdType.MESH,
            )
            r.start()
            rdmas_y.append(r)
            out_ref[pl.ds(my_j * HALF + c * R, R), :] = (
                pown_ref[pl.ds(my_j * HALF + c * R, R), :]
                + rx_ref[pl.ds(c * R, R), :]
            )

        for c in range(NCHUNK):
            rdmas_y[c].wait_recv()
            out_ref[pl.ds((1 - my_j) * HALF + c * R, R), :] = (
                pown_ref[pl.ds((1 - my_j) * HALF + c * R, R), :]
                + ry_ref[pl.ds(c * R, R), :]
            )

        for c in range(NCHUNK):
            rdmas_x[c].wait_send()
            rdmas_y[c].wait_send()

    return pl.pallas_call(
        body,
        out_shape=jax.ShapeDtypeStruct((CHUNK, F), jnp.float32),
        in_specs=[
            pl.BlockSpec(memory_space=pltpu.VMEM),
            pl.BlockSpec(memory_space=pltpu.VMEM),
        ],
        out_specs=pl.BlockSpec(memory_space=pltpu.VMEM),
        scratch_shapes=[
            pltpu.VMEM((HALF, F), jnp.float32),
            pltpu.VMEM((CHUNK, F), jnp.float32),
            pltpu.VMEM((HALF, F), jnp.float32),
            pltpu.VMEM((HALF, F), jnp.float32),
            pltpu.SemaphoreType.DMA((NCHUNK,)),
            pltpu.SemaphoreType.DMA((NCHUNK,)),
            pltpu.SemaphoreType.DMA((NCHUNK,)),
            pltpu.SemaphoreType.DMA((NCHUNK,)),
        ],
        compiler_params=pltpu.CompilerParams(collective_id=0),
    )(x, dy)


# device time: 24943 ns/iter; 1.3939x vs baseline; 1.0306x over previous
import os

import jax
import jax.numpy as jnp
from jax import lax
from jax.experimental import pallas as pl
from jax.experimental.pallas import tpu as pltpu

_VARIANT = os.environ.get("KV", "full")

M = 512
D = 512
F = 2048
HALF = D // 4
CHUNK = D // 2
NCHUNK = 4
R = HALF // NCHUNK


def _dotT(a, b):
    return lax.dot_general(
        a, b,
        dimension_numbers=(((0,), (0,)), ((), ())),
        preferred_element_type=jnp.float32,
    )


def kernel(x, dy):
    def body(x_ref, dy_ref, out_ref, psend_ref, pown_ref, rx_ref, ry_ref,
             sendx_sems, recvx_sems, sendy_sems, recvy_sems):
        my_k = lax.axis_index("x")
        my_j = lax.axis_index("y")

        if _VARIANT != "nocomm":
            barrier_sem = pltpu.get_barrier_semaphore()
            pl.semaphore_signal(
                barrier_sem, inc=1,
                device_id=(1 - my_k, my_j),
                device_id_type=pl.DeviceIdType.MESH,
            )
            pl.semaphore_signal(
                barrier_sem, inc=1,
                device_id=(my_k, 1 - my_j),
                device_id_type=pl.DeviceIdType.MESH,
            )
            pl.semaphore_wait(barrier_sem, 2)

        send_start = (1 - my_k) * CHUNK + my_j * HALF
        if _VARIANT != "nocompute":
            psend_ref[:, :] = _dotT(
                x_ref[:, pl.ds(send_start, HALF)], dy_ref[:, :]
            )

        rdmas_x = []
        for c in range(NCHUNK) if _VARIANT != "nocomm" else []:
            r = pltpu.make_async_remote_copy(
                src_ref=psend_ref.at[pl.ds(c * R, R), :],
                dst_ref=rx_ref.at[pl.ds(c * R, R), :],
                send_sem=sendx_sems.at[c],
                recv_sem=recvx_sems.at[c],
                device_id=(1 - my_k, my_j),
                device_id_type=pl.DeviceIdType.MESH,
            )
            r.start()
            rdmas_x.append(r)

        own = my_k * CHUNK
        if _VARIANT != "nocompute":
            pown_ref[:, :] = _dotT(x_ref[:, pl.ds(own, CHUNK)], dy_ref[:, :])

        rdmas_y = []
        for c in range(NCHUNK) if _VARIANT != "nocomm" else []:
            rdmas_x[c].wait_recv()
            r = pltpu.make_async_remote_copy(
                src_ref=rx_ref.at[pl.ds(c * R, R), :],
                dst_ref=ry_ref.at[pl.ds(c * R, R), :],
                send_sem=sendy_sems.at[c],
                recv_sem=recvy_sems.at[c],
                device_id=(my_k, 1 - my_j),
                device_id_type=pl.DeviceIdType.MESH,
            )
            r.start()
            rdmas_y.append(r)
            out_ref[pl.ds(my_j * HALF + c * R, R), :] = (
                pown_ref[pl.ds(my_j * HALF + c * R, R), :]
                + rx_ref[pl.ds(c * R, R), :]
            )

        for c in range(NCHUNK) if _VARIANT != "nocomm" else []:
            rdmas_y[c].wait_recv()
            out_ref[pl.ds((1 - my_j) * HALF + c * R, R), :] = (
                pown_ref[pl.ds((1 - my_j) * HALF + c * R, R), :]
                + ry_ref[pl.ds(c * R, R), :]
            )

        if _VARIANT == "nocomm":
            out_ref[pl.ds(my_j * HALF, HALF), :] = (
                pown_ref[pl.ds(my_j * HALF, HALF), :] + rx_ref[:, :]
            )
            out_ref[pl.ds((1 - my_j) * HALF, HALF), :] = (
                pown_ref[pl.ds((1 - my_j) * HALF, HALF), :] + ry_ref[:, :]
            )

        for c in range(NCHUNK) if _VARIANT != "nocomm" else []:
            rdmas_x[c].wait_send()
            rdmas_y[c].wait_send()

    return pl.pallas_call(
        body,
        out_shape=jax.ShapeDtypeStruct((CHUNK, F), jnp.float32),
        in_specs=[
            pl.BlockSpec(memory_space=pltpu.VMEM),
            pl.BlockSpec(memory_space=pltpu.VMEM),
        ],
        out_specs=pl.BlockSpec(memory_space=pltpu.VMEM),
        scratch_shapes=[
            pltpu.VMEM((HALF, F), jnp.float32),
            pltpu.VMEM((CHUNK, F), jnp.float32),
            pltpu.VMEM((HALF, F), jnp.float32),
            pltpu.VMEM((HALF, F), jnp.float32),
            pltpu.SemaphoreType.DMA((NCHUNK,)),
            pltpu.SemaphoreType.DMA((NCHUNK,)),
            pltpu.SemaphoreType.DMA((NCHUNK,)),
            pltpu.SemaphoreType.DMA((NCHUNK,)),
        ],
        compiler_params=(
            pltpu.CompilerParams(collective_id=0)
            if _VARIANT != "nocomm"
            else pltpu.CompilerParams()
        ),
    )(x, dy)


# device time: 24118 ns/iter; 1.4415x vs baseline; 1.0342x over previous
import os

import jax
import jax.numpy as jnp
from jax import lax
from jax.experimental import pallas as pl
from jax.experimental.pallas import tpu as pltpu

_VARIANT = os.environ.get("KV", "full")

M = 512
D = 512
F = 2048
HALF = D // 4
CHUNK = D // 2
NCHUNK = int(os.environ.get("KN", "8"))
R = HALF // NCHUNK
W = F // NCHUNK


def _dotT(a, b):
    return lax.dot_general(
        a, b,
        dimension_numbers=(((0,), (0,)), ((), ())),
        preferred_element_type=jnp.float32,
    )


def kernel(x, dy):
    def probe_body(x_ref, dy_ref, out_ref, psend_ref, pown_ref, rx_ref,
                   ry_ref, sendx_sems, recvx_sems, sendy_sems, recvy_sems):
        my_k = lax.axis_index("x")
        my_j = lax.axis_index("y")
        barrier_sem = pltpu.get_barrier_semaphore()
        for dev in ((1 - my_k, my_j), (my_k, 1 - my_j)):
            pl.semaphore_signal(
                barrier_sem, inc=1, device_id=dev,
                device_id_type=pl.DeviceIdType.MESH,
            )
        pl.semaphore_wait(barrier_sem, 2)
        if _VARIANT == "noop":
            out_ref[0:HALF, :] = rx_ref[:, :]
            out_ref[HALF:CHUNK, :] = ry_ref[:, :]
            return
        if _VARIANT == "uni":
            r = pltpu.make_async_remote_copy(
                src_ref=psend_ref,
                dst_ref=rx_ref,
                send_sem=sendx_sems.at[0],
                recv_sem=recvx_sems.at[0],
                device_id=(1 - my_k, my_j),
                device_id_type=pl.DeviceIdType.MESH,
            )

            @pl.when(my_k == 0)
            def _():
                r.start()
                r.wait_send()

            @pl.when(my_k == 1)
            def _():
                r.wait_recv()

            out_ref[0:HALF, :] = rx_ref[:, :]
            out_ref[HALF:CHUNK, :] = ry_ref[:, :]
            return
        if _VARIANT == "xsmall":
            r = pltpu.make_async_remote_copy(
                src_ref=psend_ref.at[pl.ds(0, 32), :],
                dst_ref=rx_ref.at[pl.ds(0, 32), :],
                send_sem=sendx_sems.at[0],
                recv_sem=recvx_sems.at[0],
                device_id=(1 - my_k, my_j),
                device_id_type=pl.DeviceIdType.MESH,
            )
            r.start()
            r.wait()
            out_ref[0:HALF, :] = rx_ref[:, :]
            out_ref[HALF:CHUNK, :] = ry_ref[:, :]
            return
        rdmas = []
        for c in range(NCHUNK):
            r = pltpu.make_async_remote_copy(
                src_ref=psend_ref.at[pl.ds(c * R, R), :],
                dst_ref=rx_ref.at[pl.ds(c * R, R), :],
                send_sem=sendx_sems.at[c],
                recv_sem=recvx_sems.at[c],
                device_id=(1 - my_k, my_j),
                device_id_type=pl.DeviceIdType.MESH,
            )
            r.start()
            rdmas.append(r)
        if _VARIANT == "dual":
            for c in range(NCHUNK):
                r = pltpu.make_async_remote_copy(
                    src_ref=psend_ref.at[pl.ds(c * R, R), :],
                    dst_ref=ry_ref.at[pl.ds(c * R, R), :],
                    send_sem=sendy_sems.at[c],
                    recv_sem=recvy_sems.at[c],
                    device_id=(my_k, 1 - my_j),
                    device_id_type=pl.DeviceIdType.MESH,
                )
                r.start()
                rdmas.append(r)
        for r in rdmas:
            r.wait()
        out_ref[0:HALF, :] = rx_ref[:, :]
        out_ref[HALF:CHUNK, :] = ry_ref[:, :]

    def body(x_ref, dy_ref, out_ref, psend_ref, pown_ref, rx_ref, ry_ref,
             sendx_sems, recvx_sems, sendy_sems, recvy_sems):
        my_k = lax.axis_index("x")
        my_j = lax.axis_index("y")

        if _VARIANT != "nocomm":
            barrier_sem = pltpu.get_barrier_semaphore()
            pl.semaphore_signal(
                barrier_sem, inc=1,
                device_id=(1 - my_k, my_j),
                device_id_type=pl.DeviceIdType.MESH,
            )
            pl.semaphore_signal(
                barrier_sem, inc=1,
                device_id=(my_k, 1 - my_j),
                device_id_type=pl.DeviceIdType.MESH,
            )
            pl.semaphore_wait(barrier_sem, 2)

        send_start = (1 - my_k) * CHUNK + my_j * HALF
        x_send_blk = x_ref[:, pl.ds(send_start, HALF)]
        rdmas_x = []
        for c in range(NCHUNK) if _VARIANT != "nocomm" else []:
            if _VARIANT != "nocompute":
                psend_ref[:, c * W:(c + 1) * W] = _dotT(
                    x_send_blk, dy_ref[:, c * W:(c + 1) * W]
                )
            r = pltpu.make_async_remote_copy(
                src_ref=psend_ref.at[:, pl.ds(c * W, W)],
                dst_ref=rx_ref.at[:, pl.ds(c * W, W)],
                send_sem=sendx_sems.at[c],
                recv_sem=recvx_sems.at[c],
                device_id=(1 - my_k, my_j),
                device_id_type=pl.DeviceIdType.MESH,
            )
            r.start()
            rdmas_x.append(r)
        if _VARIANT == "nocomm":
            psend_ref[:, :] = _dotT(x_send_blk, dy_ref[:, :])

        own = my_k * CHUNK
        if _VARIANT != "nocompute":
            pown_ref[pl.ds(my_j * HALF, HALF), :] = _dotT(
                x_ref[:, pl.ds(own + my_j * HALF, HALF)], dy_ref[:, :]
            )

        rdmas_y = []
        for c in range(NCHUNK) if _VARIANT != "nocomm" else []:
            rdmas_x[c].wait_recv()
            r = pltpu.make_async_remote_copy(
                src_ref=rx_ref.at[:, pl.ds(c * W, W)],
                dst_ref=ry_ref.at[:, pl.ds(c * W, W)],
                send_sem=sendy_sems.at[c],
                recv_sem=recvy_sems.at[c],
                device_id=(my_k, 1 - my_j),
                device_id_type=pl.DeviceIdType.MESH,
            )
            r.start()
            rdmas_y.append(r)
            out_ref[pl.ds(my_j * HALF, HALF), c * W:(c + 1) * W] = (
                pown_ref[pl.ds(my_j * HALF, HALF), c * W:(c + 1) * W]
                + rx_ref[:, c * W:(c + 1) * W]
            )

        if _VARIANT != "nocompute":
            pown_ref[pl.ds((1 - my_j) * HALF, HALF), :] = _dotT(
                x_ref[:, pl.ds(own + (1 - my_j) * HALF, HALF)],
                dy_ref[:, :],
            )

        for c in range(NCHUNK) if _VARIANT != "nocomm" else []:
            rdmas_y[c].wait_recv()
            out_ref[pl.ds((1 - my_j) * HALF, HALF), c * W:(c + 1) * W] = (
                pown_ref[pl.ds((1 - my_j) * HALF, HALF), c * W:(c + 1) * W]
                + ry_ref[:, c * W:(c + 1) * W]
            )

        if _VARIANT == "nocomm":
            out_ref[pl.ds(my_j * HALF, HALF), :] = (
                pown_ref[pl.ds(my_j * HALF, HALF), :] + rx_ref[:, :]
            )
            out_ref[pl.ds((1 - my_j) * HALF, HALF), :] = (
                pown_ref[pl.ds((1 - my_j) * HALF, HALF), :] + ry_ref[:, :]
            )

        for c in range(NCHUNK) if _VARIANT != "nocomm" else []:
            rdmas_x[c].wait_send()
            rdmas_y[c].wait_send()

    return pl.pallas_call(
        probe_body
        if _VARIANT in ("xonly", "dual", "uni", "xsmall", "noop")
        else body,
        out_shape=jax.ShapeDtypeStruct((CHUNK, F), jnp.float32),
        in_specs=[
            pl.BlockSpec(memory_space=pltpu.VMEM),
            pl.BlockSpec(memory_space=pltpu.VMEM),
        ],
        out_specs=pl.BlockSpec(memory_space=pltpu.VMEM),
        scratch_shapes=[
            pltpu.VMEM((HALF, F), jnp.float32),
            pltpu.VMEM((CHUNK, F), jnp.float32),
            pltpu.VMEM((HALF, F), jnp.float32),
            pltpu.VMEM((HALF, F), jnp.float32),
            pltpu.SemaphoreType.DMA((NCHUNK,)),
            pltpu.SemaphoreType.DMA((NCHUNK,)),
            pltpu.SemaphoreType.DMA((NCHUNK,)),
            pltpu.SemaphoreType.DMA((NCHUNK,)),
        ],
        compiler_params=(
            pltpu.CompilerParams(collective_id=0)
            if _VARIANT != "nocomm"
            else pltpu.CompilerParams()
        ),
    )(x, dy)


# device time: 24009 ns/iter; 1.4481x vs baseline; 1.0045x over previous
import os

import jax
import jax.numpy as jnp
from jax import lax
from jax.experimental import pallas as pl
from jax.experimental.pallas import tpu as pltpu

_VARIANT = os.environ.get("KV", "full")

M = 512
D = 512
F = 2048
HALF = D // 4
CHUNK = D // 2
NCHUNK = int(os.environ.get("KN", "8"))
R = HALF // NCHUNK
W = F // NCHUNK


def _dotT(a, b):
    return lax.dot_general(
        a, b,
        dimension_numbers=(((0,), (0,)), ((), ())),
        preferred_element_type=jnp.float32,
    )


def kernel(x, dy):
    def probe_body(x_ref, dy_ref, out_ref, psend_ref, pown_ref, rx_ref,
                   ry_ref, sendx_sems, recvx_sems, sendy_sems, recvy_sems):
        my_k = lax.axis_index("x")
        my_j = lax.axis_index("y")
        barrier_sem = pltpu.get_barrier_semaphore()
        for dev in ((1 - my_k, my_j), (my_k, 1 - my_j)):
            pl.semaphore_signal(
                barrier_sem, inc=1, device_id=dev,
                device_id_type=pl.DeviceIdType.MESH,
            )
        pl.semaphore_wait(barrier_sem, 2)
        if _VARIANT == "noop":
            out_ref[0:HALF, :] = rx_ref[:, :]
            out_ref[HALF:CHUNK, :] = ry_ref[:, :]
            return
        if _VARIANT == "uni":
            r = pltpu.make_async_remote_copy(
                src_ref=psend_ref,
                dst_ref=rx_ref,
                send_sem=sendx_sems.at[0],
                recv_sem=recvx_sems.at[0],
                device_id=(1 - my_k, my_j),
                device_id_type=pl.DeviceIdType.MESH,
            )

            @pl.when(my_k == 0)
            def _():
                r.start()
                r.wait_send()

            @pl.when(my_k == 1)
            def _():
                r.wait_recv()

            out_ref[0:HALF, :] = rx_ref[:, :]
            out_ref[HALF:CHUNK, :] = ry_ref[:, :]
            return
        if _VARIANT == "xsmall":
            r = pltpu.make_async_remote_copy(
                src_ref=psend_ref.at[pl.ds(0, 32), :],
                dst_ref=rx_ref.at[pl.ds(0, 32), :],
                send_sem=sendx_sems.at[0],
                recv_sem=recvx_sems.at[0],
                device_id=(1 - my_k, my_j),
                device_id_type=pl.DeviceIdType.MESH,
            )
            r.start()
            r.wait()
            out_ref[0:HALF, :] = rx_ref[:, :]
            out_ref[HALF:CHUNK, :] = ry_ref[:, :]
            return
        rdmas = []
        for c in range(NCHUNK):
            r = pltpu.make_async_remote_copy(
                src_ref=psend_ref.at[pl.ds(c * R, R), :],
                dst_ref=rx_ref.at[pl.ds(c * R, R), :],
                send_sem=sendx_sems.at[c],
                recv_sem=recvx_sems.at[c],
                device_id=(1 - my_k, my_j),
                device_id_type=pl.DeviceIdType.MESH,
            )
            r.start()
            rdmas.append(r)
        if _VARIANT == "dual":
            for c in range(NCHUNK):
                r = pltpu.make_async_remote_copy(
                    src_ref=psend_ref.at[pl.ds(c * R, R), :],
                    dst_ref=ry_ref.at[pl.ds(c * R, R), :],
                    send_sem=sendy_sems.at[c],
                    recv_sem=recvy_sems.at[c],
                    device_id=(my_k, 1 - my_j),
                    device_id_type=pl.DeviceIdType.MESH,
                )
                r.start()
                rdmas.append(r)
        for r in rdmas:
            r.wait()
        out_ref[0:HALF, :] = rx_ref[:, :]
        out_ref[HALF:CHUNK, :] = ry_ref[:, :]

    def body(x_ref, dy_ref, out_ref, psend_ref, pown_ref, rx_ref, ry_ref,
             sendx_sems, recvx_sems, sendy_sems, recvy_sems):
        my_k = lax.axis_index("x")
        my_j = lax.axis_index("y")

        if _VARIANT != "nocomm":
            barrier_sem = pltpu.get_barrier_semaphore()
            pl.semaphore_signal(
                barrier_sem, inc=1,
                device_id=(1 - my_k, my_j),
                device_id_type=pl.DeviceIdType.MESH,
            )
            pl.semaphore_signal(
                barrier_sem, inc=1,
                device_id=(my_k, 1 - my_j),
                device_id_type=pl.DeviceIdType.MESH,
            )
            pl.semaphore_wait(barrier_sem, 2)

        send_start = (1 - my_k) * CHUNK + my_j * HALF
        x_send_blk = x_ref[:, pl.ds(send_start, HALF)]
        rdmas_x = []
        for c in range(NCHUNK) if _VARIANT != "nocomm" else []:
            if _VARIANT != "nocompute":
                psend_ref[:, c * W:(c + 1) * W] = _dotT(
                    x_send_blk, dy_ref[:, c * W:(c + 1) * W]
                )
            r = pltpu.make_async_remote_copy(
                src_ref=psend_ref.at[:, pl.ds(c * W, W)],
                dst_ref=rx_ref.at[:, pl.ds(c * W, W)],
                send_sem=sendx_sems.at[c],
                recv_sem=recvx_sems.at[c],
                device_id=(1 - my_k, my_j),
                device_id_type=pl.DeviceIdType.MESH,
            )
            r.start()
            rdmas_x.append(r)
        if _VARIANT == "nocomm":
            psend_ref[:, :] = _dotT(x_send_blk, dy_ref[:, :])

        own = my_k * CHUNK
        x_own_j = x_ref[:, pl.ds(own + my_j * HALF, HALF)]
        x_own_o = x_ref[:, pl.ds(own + (1 - my_j) * HALF, HALF)]
        rdmas_y = []
        for c in range(NCHUNK) if _VARIANT != "nocomm" else []:
            rdmas_x[c].wait_recv()
            r = pltpu.make_async_remote_copy(
                src_ref=rx_ref.at[:, pl.ds(c * W, W)],
                dst_ref=ry_ref.at[:, pl.ds(c * W, W)],
                send_sem=sendy_sems.at[c],
                recv_sem=recvy_sems.at[c],
                device_id=(my_k, 1 - my_j),
                device_id_type=pl.DeviceIdType.MESH,
            )
            r.start()
            rdmas_y.append(r)
            if _VARIANT != "nocompute":
                out_ref[pl.ds(my_j * HALF, HALF), c * W:(c + 1) * W] = (
                    _dotT(x_own_j, dy_ref[:, c * W:(c + 1) * W])
                    + rx_ref[:, c * W:(c + 1) * W]
                )
            else:
                out_ref[pl.ds(my_j * HALF, HALF), c * W:(c + 1) * W] = (
                    rx_ref[:, c * W:(c + 1) * W]
                )

        for c in range(NCHUNK) if _VARIANT != "nocomm" else []:
            rdmas_y[c].wait_recv()
            if _VARIANT != "nocompute":
                out_ref[pl.ds((1 - my_j) * HALF, HALF), c * W:(c + 1) * W] = (
                    _dotT(x_own_o, dy_ref[:, c * W:(c + 1) * W])
                    + ry_ref[:, c * W:(c + 1) * W]
                )
            else:
                out_ref[pl.ds((1 - my_j) * HALF, HALF), c * W:(c + 1) * W] = (
                    ry_ref[:, c * W:(c + 1) * W]
                )

        if _VARIANT == "nocomm":
            out_ref[pl.ds(my_j * HALF, HALF), :] = (
                _dotT(x_own_j, dy_ref[:, :]) + rx_ref[:, :]
            )
            out_ref[pl.ds((1 - my_j) * HALF, HALF), :] = (
                _dotT(x_own_o, dy_ref[:, :]) + ry_ref[:, :]
            )

        for c in range(NCHUNK) if _VARIANT != "nocomm" else []:
            rdmas_x[c].wait_send()
            rdmas_y[c].wait_send()

    return pl.pallas_call(
        probe_body
        if _VARIANT in ("xonly", "dual", "uni", "xsmall", "noop")
        else body,
        out_shape=jax.ShapeDtypeStruct((CHUNK, F), jnp.float32),
        in_specs=[
            pl.BlockSpec(memory_space=pltpu.VMEM),
            pl.BlockSpec(memory_space=pltpu.VMEM),
        ],
        out_specs=pl.BlockSpec(memory_space=pltpu.VMEM),
        scratch_shapes=[
            pltpu.VMEM((HALF, F), jnp.float32),
            pltpu.VMEM((CHUNK, F), jnp.float32),
            pltpu.VMEM((HALF, F), jnp.float32),
            pltpu.VMEM((HALF, F), jnp.float32),
            pltpu.SemaphoreType.DMA((NCHUNK,)),
            pltpu.SemaphoreType.DMA((NCHUNK,)),
            pltpu.SemaphoreType.DMA((NCHUNK,)),
            pltpu.SemaphoreType.DMA((NCHUNK,)),
        ],
        compiler_params=(
            pltpu.CompilerParams(collective_id=0)
            if _VARIANT != "nocomm"
            else pltpu.CompilerParams()
        ),
    )(x, dy)


# device time: 17662 ns/iter; 1.9685x vs baseline; 1.3594x over previous
import os

import jax
import jax.numpy as jnp
from jax import lax
from jax.experimental import pallas as pl
from jax.experimental.pallas import tpu as pltpu

_VARIANT = os.environ.get("KV", "full")

M = 512
D = 512
F = 2048
HALF = D // 4
CHUNK = D // 2
NCHUNK = int(os.environ.get("KN", "8"))
R = HALF // NCHUNK
W = F // NCHUNK


def _dotT(a, b):
    return lax.dot_general(
        a, b,
        dimension_numbers=(((0,), (0,)), ((), ())),
        preferred_element_type=jnp.float32,
    )


def kernel(x, dy):
    def probe_body(x_ref, dy_ref, out_ref, psend_ref, pown_ref, rx_ref,
                   ry_ref, sendx_sems, recvx_sems, sendy_sems, recvy_sems):
        my_k = lax.axis_index("x")
        my_j = lax.axis_index("y")
        barrier_sem = pltpu.get_barrier_semaphore()
        for dev in ((1 - my_k, my_j), (my_k, 1 - my_j)):
            pl.semaphore_signal(
                barrier_sem, inc=1, device_id=dev,
                device_id_type=pl.DeviceIdType.MESH,
            )
        pl.semaphore_wait(barrier_sem, 2)
        if _VARIANT == "noop":
            out_ref[0:HALF, :] = rx_ref[:, :].astype(jnp.float32)
            out_ref[HALF:CHUNK, :] = ry_ref[:, :].astype(jnp.float32)
            return
        if _VARIANT == "uni":
            r = pltpu.make_async_remote_copy(
                src_ref=psend_ref,
                dst_ref=rx_ref,
                send_sem=sendx_sems.at[0],
                recv_sem=recvx_sems.at[0],
                device_id=(1 - my_k, my_j),
                device_id_type=pl.DeviceIdType.MESH,
            )

            @pl.when(my_k == 0)
            def _():
                r.start()
                r.wait_send()

            @pl.when(my_k == 1)
            def _():
                r.wait_recv()

            out_ref[0:HALF, :] = rx_ref[:, :].astype(jnp.float32)
            out_ref[HALF:CHUNK, :] = ry_ref[:, :].astype(jnp.float32)
            return
        if _VARIANT == "xsmall":
            r = pltpu.make_async_remote_copy(
                src_ref=psend_ref.at[pl.ds(0, 32), :],
                dst_ref=rx_ref.at[pl.ds(0, 32), :],
                send_sem=sendx_sems.at[0],
                recv_sem=recvx_sems.at[0],
                device_id=(1 - my_k, my_j),
                device_id_type=pl.DeviceIdType.MESH,
            )
            r.start()
            r.wait()
            out_ref[0:HALF, :] = rx_ref[:, :].astype(jnp.float32)
            out_ref[HALF:CHUNK, :] = ry_ref[:, :].astype(jnp.float32)
            return
        rdmas = []
        for c in range(NCHUNK):
            r = pltpu.make_async_remote_copy(
                src_ref=psend_ref.at[pl.ds(c * R, R), :],
                dst_ref=rx_ref.at[pl.ds(c * R, R), :],
                send_sem=sendx_sems.at[c],
                recv_sem=recvx_sems.at[c],
                device_id=(1 - my_k, my_j),
                device_id_type=pl.DeviceIdType.MESH,
            )
            r.start()
            rdmas.append(r)
        if _VARIANT == "dual":
            for c in range(NCHUNK):
                r = pltpu.make_async_remote_copy(
                    src_ref=psend_ref.at[pl.ds(c * R, R), :],
                    dst_ref=ry_ref.at[pl.ds(c * R, R), :],
                    send_sem=sendy_sems.at[c],
                    recv_sem=recvy_sems.at[c],
                    device_id=(my_k, 1 - my_j),
                    device_id_type=pl.DeviceIdType.MESH,
                )
                r.start()
                rdmas.append(r)
        for r in rdmas:
            r.wait()
        out_ref[0:HALF, :] = rx_ref[:, :].astype(jnp.float32)
        out_ref[HALF:CHUNK, :] = ry_ref[:, :].astype(jnp.float32)

    def body(x_ref, dy_ref, out_ref, psend_ref, pown_ref, rx_ref, ry_ref,
             sendx_sems, recvx_sems, sendy_sems, recvy_sems):
        my_k = lax.axis_index("x")
        my_j = lax.axis_index("y")

        if _VARIANT != "nocomm":
            barrier_sem = pltpu.get_barrier_semaphore()
            pl.semaphore_signal(
                barrier_sem, inc=1,
                device_id=(1 - my_k, my_j),
                device_id_type=pl.DeviceIdType.MESH,
            )
            pl.semaphore_signal(
                barrier_sem, inc=1,
                device_id=(my_k, 1 - my_j),
                device_id_type=pl.DeviceIdType.MESH,
            )

        send_start = (1 - my_k) * CHUNK + my_j * HALF
        x_send_blk = x_ref[:, pl.ds(send_start, HALF)]
        rdmas_x = []
        for c in range(NCHUNK) if _VARIANT != "nocomm" else []:
            if _VARIANT != "nocompute":
                psend_ref[:, c * W:(c + 1) * W] = _dotT(
                    x_send_blk, dy_ref[:, c * W:(c + 1) * W]
                ).astype(jnp.bfloat16)
            if c == 0:
                pl.semaphore_wait(barrier_sem, 2)
            r = pltpu.make_async_remote_copy(
                src_ref=psend_ref.at[:, pl.ds(c * W, W)],
                dst_ref=rx_ref.at[:, pl.ds(c * W, W)],
                send_sem=sendx_sems.at[c],
                recv_sem=recvx_sems.at[c],
                device_id=(1 - my_k, my_j),
                device_id_type=pl.DeviceIdType.MESH,
            )
            r.start()
            rdmas_x.append(r)
        if _VARIANT == "nocomm":
            psend_ref[:, :] = _dotT(
                x_send_blk, dy_ref[:, :]
            ).astype(jnp.bfloat16)

        own = my_k * CHUNK
        x_own_j = x_ref[:, pl.ds(own + my_j * HALF, HALF)]
        x_own_o = x_ref[:, pl.ds(own + (1 - my_j) * HALF, HALF)]
        rdmas_y = []
        for c in range(NCHUNK) if _VARIANT != "nocomm" else []:
            rdmas_x[c].wait_recv()
            r = pltpu.make_async_remote_copy(
                src_ref=rx_ref.at[:, pl.ds(c * W, W)],
                dst_ref=ry_ref.at[:, pl.ds(c * W, W)],
                send_sem=sendy_sems.at[c],
                recv_sem=recvy_sems.at[c],
                device_id=(my_k, 1 - my_j),
                device_id_type=pl.DeviceIdType.MESH,
            )
            r.start()
            rdmas_y.append(r)
            if _VARIANT != "nocompute":
                out_ref[pl.ds(my_j * HALF, HALF), c * W:(c + 1) * W] = (
                    _dotT(x_own_j, dy_ref[:, c * W:(c + 1) * W])
                    + rx_ref[:, c * W:(c + 1) * W].astype(jnp.float32)
                )
            else:
                out_ref[pl.ds(my_j * HALF, HALF), c * W:(c + 1) * W] = (
                    rx_ref[:, c * W:(c + 1) * W].astype(jnp.float32)
                )

        for c in range(NCHUNK) if _VARIANT != "nocomm" else []:
            rdmas_y[c].wait_recv()
            if _VARIANT != "nocompute":
                out_ref[pl.ds((1 - my_j) * HALF, HALF), c * W:(c + 1) * W] = (
                    _dotT(x_own_o, dy_ref[:, c * W:(c + 1) * W])
                    + ry_ref[:, c * W:(c + 1) * W].astype(jnp.float32)
                )
            else:
                out_ref[pl.ds((1 - my_j) * HALF, HALF), c * W:(c + 1) * W] = (
                    ry_ref[:, c * W:(c + 1) * W].astype(jnp.float32)
                )

        if _VARIANT == "nocomm":
            out_ref[pl.ds(my_j * HALF, HALF), :] = (
                _dotT(x_own_j, dy_ref[:, :]) + rx_ref[:, :].astype(jnp.float32)
            )
            out_ref[pl.ds((1 - my_j) * HALF, HALF), :] = (
                _dotT(x_own_o, dy_ref[:, :]) + ry_ref[:, :].astype(jnp.float32)
            )

        for c in range(NCHUNK) if _VARIANT != "nocomm" else []:
            rdmas_x[c].wait_send()
            rdmas_y[c].wait_send()

    return pl.pallas_call(
        probe_body
        if _VARIANT in ("xonly", "dual", "uni", "xsmall", "noop")
        else body,
        out_shape=jax.ShapeDtypeStruct((CHUNK, F), jnp.float32),
        in_specs=[
            pl.BlockSpec(memory_space=pltpu.VMEM),
            pl.BlockSpec(memory_space=pltpu.VMEM),
        ],
        out_specs=pl.BlockSpec(memory_space=pltpu.VMEM),
        scratch_shapes=[
            pltpu.VMEM((HALF, F), jnp.bfloat16),
            pltpu.VMEM((CHUNK, F), jnp.float32),
            pltpu.VMEM((HALF, F), jnp.bfloat16),
            pltpu.VMEM((HALF, F), jnp.bfloat16),
            pltpu.SemaphoreType.DMA((NCHUNK,)),
            pltpu.SemaphoreType.DMA((NCHUNK,)),
            pltpu.SemaphoreType.DMA((NCHUNK,)),
            pltpu.SemaphoreType.DMA((NCHUNK,)),
        ],
        compiler_params=(
            pltpu.CompilerParams(collective_id=0)
            if _VARIANT != "nocomm"
            else pltpu.CompilerParams()
        ),
    )(x, dy)


# device time: 17624 ns/iter; 1.9727x vs baseline; 1.0022x over previous
import os

import jax
import jax.numpy as jnp
from jax import lax
from jax.experimental import pallas as pl
from jax.experimental.pallas import tpu as pltpu

_VARIANT = os.environ.get("KV", "full")

M = 512
D = 512
F = 2048
HALF = D // 4
CHUNK = D // 2
NCHUNK = int(os.environ.get("KN", "8"))
R = HALF // NCHUNK
W = F // NCHUNK


def _dotT(a, b):
    return lax.dot_general(
        a, b,
        dimension_numbers=(((0,), (0,)), ((), ())),
        preferred_element_type=jnp.float32,
    )


def kernel(x, dy):
    def probe_body(x_ref, dy_ref, out_ref, psend_ref, pown_ref, rx_ref,
                   ry_ref, sendx_sems, recvx_sems, sendy_sems, recvy_sems):
        my_k = lax.axis_index("x")
        my_j = lax.axis_index("y")
        barrier_sem = pltpu.get_barrier_semaphore()
        for dev in ((1 - my_k, my_j), (my_k, 1 - my_j)):
            pl.semaphore_signal(
                barrier_sem, inc=1, device_id=dev,
                device_id_type=pl.DeviceIdType.MESH,
            )
        pl.semaphore_wait(barrier_sem, 2)
        if _VARIANT == "noop":
            out_ref[0:HALF, :] = rx_ref[:, :].astype(jnp.float32)
            out_ref[HALF:CHUNK, :] = ry_ref[:, :].astype(jnp.float32)
            return
        if _VARIANT == "bar":
            out_ref[0:8, :] = rx_ref[0:8, :].astype(jnp.float32)
            return
        if _VARIANT == "uni":
            r = pltpu.make_async_remote_copy(
                src_ref=psend_ref,
                dst_ref=rx_ref,
                send_sem=sendx_sems.at[0],
                recv_sem=recvx_sems.at[0],
                device_id=(1 - my_k, my_j),
                device_id_type=pl.DeviceIdType.MESH,
            )

            @pl.when(my_k == 0)
            def _():
                r.start()
                r.wait_send()

            @pl.when(my_k == 1)
            def _():
                r.wait_recv()

            out_ref[0:HALF, :] = rx_ref[:, :].astype(jnp.float32)
            out_ref[HALF:CHUNK, :] = ry_ref[:, :].astype(jnp.float32)
            return
        if _VARIANT == "xsmall":
            r = pltpu.make_async_remote_copy(
                src_ref=psend_ref.at[pl.ds(0, 32), :],
                dst_ref=rx_ref.at[pl.ds(0, 32), :],
                send_sem=sendx_sems.at[0],
                recv_sem=recvx_sems.at[0],
                device_id=(1 - my_k, my_j),
                device_id_type=pl.DeviceIdType.MESH,
            )
            r.start()
            r.wait()
            out_ref[0:HALF, :] = rx_ref[:, :].astype(jnp.float32)
            out_ref[HALF:CHUNK, :] = ry_ref[:, :].astype(jnp.float32)
            return
        rdmas = []
        for c in range(NCHUNK):
            r = pltpu.make_async_remote_copy(
                src_ref=psend_ref.at[pl.ds(c * R, R), :],
                dst_ref=rx_ref.at[pl.ds(c * R, R), :],
                send_sem=sendx_sems.at[c],
                recv_sem=recvx_sems.at[c],
                device_id=(1 - my_k, my_j),
                device_id_type=pl.DeviceIdType.MESH,
            )
            r.start()
            rdmas.append(r)
        if _VARIANT == "dual":
            for c in range(NCHUNK):
                r = pltpu.make_async_remote_copy(
                    src_ref=psend_ref.at[pl.ds(c * R, R), :],
                    dst_ref=ry_ref.at[pl.ds(c * R, R), :],
                    send_sem=sendy_sems.at[c],
                    recv_sem=recvy_sems.at[c],
                    device_id=(my_k, 1 - my_j),
                    device_id_type=pl.DeviceIdType.MESH,
                )
                r.start()
                rdmas.append(r)
        for r in rdmas:
            r.wait()
        out_ref[0:HALF, :] = rx_ref[:, :].astype(jnp.float32)
        out_ref[HALF:CHUNK, :] = ry_ref[:, :].astype(jnp.float32)

    def body(x_ref, dy_ref, out_ref, psend_ref, pown_ref, rx_ref, ry_ref,
             sendx_sems, recvx_sems, sendy_sems, recvy_sems):
        my_k = lax.axis_index("x")
        my_j = lax.axis_index("y")

        if _VARIANT != "nocomm":
            barrier_sem = pltpu.get_barrier_semaphore()
            pl.semaphore_signal(
                barrier_sem, inc=1,
                device_id=(1 - my_k, my_j),
                device_id_type=pl.DeviceIdType.MESH,
            )
            pl.semaphore_signal(
                barrier_sem, inc=1,
                device_id=(my_k, 1 - my_j),
                device_id_type=pl.DeviceIdType.MESH,
            )

        send_start = (1 - my_k) * CHUNK + my_j * HALF
        x_send_blk = x_ref[:, pl.ds(send_start, HALF)]
        own = my_k * CHUNK
        x_own_j = x_ref[:, pl.ds(own + my_j * HALF, HALF)]
        x_own_o = x_ref[:, pl.ds(own + (1 - my_j) * HALF, HALF)]
        rdmas_x = []
        for c in range(NCHUNK) if _VARIANT != "nocomm" else []:
            if _VARIANT != "nocompute":
                psend_ref[:, c * W:(c + 1) * W] = _dotT(
                    x_send_blk, dy_ref[:, c * W:(c + 1) * W]
                ).astype(jnp.bfloat16)
            if c == 0:
                pl.semaphore_wait(barrier_sem, 2)
            r = pltpu.make_async_remote_copy(
                src_ref=psend_ref.at[:, pl.ds(c * W, W)],
                dst_ref=rx_ref.at[:, pl.ds(c * W, W)],
                send_sem=sendx_sems.at[c],
                recv_sem=recvx_sems.at[c],
                device_id=(1 - my_k, my_j),
                device_id_type=pl.DeviceIdType.MESH,
            )
            r.start()
            rdmas_x.append(r)
            if _VARIANT != "nocompute":
                pown_ref[0:HALF, c * W:(c + 1) * W] = _dotT(
                    x_own_o, dy_ref[:, c * W:(c + 1) * W]
                )
        if _VARIANT == "nocomm":
            psend_ref[:, :] = _dotT(
                x_send_blk, dy_ref[:, :]
            ).astype(jnp.bfloat16)

        rdmas_y = []
        for c in range(NCHUNK) if _VARIANT != "nocomm" else []:
            rdmas_x[c].wait_recv()
            r = pltpu.make_async_remote_copy(
                src_ref=rx_ref.at[:, pl.ds(c * W, W)],
                dst_ref=ry_ref.at[:, pl.ds(c * W, W)],
                send_sem=sendy_sems.at[c],
                recv_sem=recvy_sems.at[c],
                device_id=(my_k, 1 - my_j),
                device_id_type=pl.DeviceIdType.MESH,
            )
            r.start()
            rdmas_y.append(r)
            if _VARIANT != "nocompute":
                out_ref[pl.ds(my_j * HALF, HALF), c * W:(c + 1) * W] = (
                    _dotT(x_own_j, dy_ref[:, c * W:(c + 1) * W])
                    + rx_ref[:, c * W:(c + 1) * W].astype(jnp.float32)
                )
            else:
                out_ref[pl.ds(my_j * HALF, HALF), c * W:(c + 1) * W] = (
                    rx_ref[:, c * W:(c + 1) * W].astype(jnp.float32)
                )

        for c in range(NCHUNK) if _VARIANT != "nocomm" else []:
            rdmas_y[c].wait_recv()
            if _VARIANT != "nocompute":
                out_ref[pl.ds((1 - my_j) * HALF, HALF), c * W:(c + 1) * W] = (
                    pown_ref[0:HALF, c * W:(c + 1) * W]
                    + ry_ref[:, c * W:(c + 1) * W].astype(jnp.float32)
                )
            else:
                out_ref[pl.ds((1 - my_j) * HALF, HALF), c * W:(c + 1) * W] = (
                    ry_ref[:, c * W:(c + 1) * W].astype(jnp.float32)
                )

        if _VARIANT == "nocomm":
            out_ref[pl.ds(my_j * HALF, HALF), :] = (
                _dotT(x_own_j, dy_ref[:, :]) + rx_ref[:, :].astype(jnp.float32)
            )
            out_ref[pl.ds((1 - my_j) * HALF, HALF), :] = (
                _dotT(x_own_o, dy_ref[:, :]) + ry_ref[:, :].astype(jnp.float32)
            )

        for c in range(NCHUNK) if _VARIANT != "nocomm" else []:
            rdmas_x[c].wait_send()
            rdmas_y[c].wait_send()

    return pl.pallas_call(
        probe_body
        if _VARIANT in ("xonly", "dual", "uni", "xsmall", "noop", "bar")
        else body,
        out_shape=jax.ShapeDtypeStruct((CHUNK, F), jnp.float32),
        in_specs=[
            pl.BlockSpec(memory_space=pltpu.VMEM),
            pl.BlockSpec(memory_space=pltpu.VMEM),
        ],
        out_specs=pl.BlockSpec(memory_space=pltpu.VMEM),
        scratch_shapes=[
            pltpu.VMEM((HALF, F), jnp.bfloat16),
            pltpu.VMEM((CHUNK, F), jnp.float32),
            pltpu.VMEM((HALF, F), jnp.bfloat16),
            pltpu.VMEM((HALF, F), jnp.bfloat16),
            pltpu.SemaphoreType.DMA((NCHUNK,)),
            pltpu.SemaphoreType.DMA((NCHUNK,)),
            pltpu.SemaphoreType.DMA((NCHUNK,)),
            pltpu.SemaphoreType.DMA((NCHUNK,)),
        ],
        compiler_params=(
            pltpu.CompilerParams(collective_id=0)
            if _VARIANT != "nocomm"
            else pltpu.CompilerParams()
        ),
    )(x, dy)
